# baseline (speedup 1.0000x reference)
import functools

import jax
import jax.numpy as jnp
from jax import lax
from jax.experimental import pallas as pl
from jax.experimental.pallas import tpu as pltpu
from jax.experimental.pallas import tpu_sc as plsc

NC, NS, L = 2, 16, 16
NW = NC * NS
B = 16384
F = 16
BPW = B // NW
G = BPW // L
BL = BPW * F


def _lfm_body(users, items, ub_hbm, ib_hbm, ue_hbm, ie_hbm, out_hbm,
              idx_u, idx_i, ue_s, ie_s, ub_s, ib_s, out_s,
              sem_u, sem_i, sem_ub, sem_ib):
  wid = lax.axis_index("s") * NC + lax.axis_index("c")
  base = wid * BPW

  pltpu.sync_copy(users.at[pl.ds(base, BPW)], idx_u)
  pltpu.sync_copy(items.at[pl.ds(base, BPW)], idx_i)

  cu = pltpu.async_copy(ue_hbm.at[idx_u], ue_s, sem_u)
  ci = pltpu.async_copy(ie_hbm.at[idx_i], ie_s, sem_i)

  def fire(j, carry):
    uu = idx_u[pl.ds(j * L, L)]
    vv = idx_i[pl.ds(j * L, L)]
    for k in range(L):
      i = j * L + k
      u = uu[k]
      v = vv[k]
      pltpu.async_copy(ub_hbm.at[0, pl.ds((u >> 4) * F, F)],
                       ub_s.at[pl.ds(i * F, F)], sem_ub)
      pltpu.async_copy(ib_hbm.at[0, pl.ds((v >> 4) * F, F)],
                       ib_s.at[pl.ds(i * F, F)], sem_ib)
    return carry

  lax.fori_loop(0, G, fire, 0)
  pltpu.make_async_copy(ub_hbm.at[0, pl.ds(0, BL)], ub_s, sem_ub).wait()
  pltpu.make_async_copy(ib_hbm.at[0, pl.ds(0, BL)], ib_s, sem_ib).wait()
  cu.wait()
  ci.wait()

  lane = lax.iota(jnp.int32, L)
  zero = jnp.zeros((L,), jnp.int32)

  def group(g, carry):
    off = g * L
    rows = off + lane
    uu = idx_u[pl.ds(off, L)]
    vv = idx_i[pl.ds(off, L)]
    fbase = rows * F
    acc = (plsc.load_gather(ub_s, [fbase + (uu & 15)]) +
           plsc.load_gather(ib_s, [fbase + (vv & 15)]))
    for f in range(F):
      col = jnp.full((L,), f, jnp.int32)
      acc = acc + (plsc.load_gather(ue_s, [rows, col]) *
                   plsc.load_gather(ie_s, [rows, col]))
    out_s[pl.ds(off, L)] = acc
    return carry

  lax.fori_loop(0, G, group, 0)
  pltpu.sync_copy(out_s, out_hbm.at[pl.ds(base, BPW)])


@functools.partial(jax.jit, static_argnames=())
def _lfm(users, items, ub, ib, ue, ie):
  mesh = plsc.VectorSubcoreMesh(
      core_axis_name="c", subcore_axis_name="s",
      num_cores=NC, num_subcores=NS)
  run = pl.kernel(
      _lfm_body,
      out_type=jax.ShapeDtypeStruct((B,), jnp.float32),
      mesh=mesh,
      compiler_params=pltpu.CompilerParams(needs_layout_passes=False,
                                           use_tc_tiling_on_sc=False),
      scratch_types=[
          pltpu.VMEM((BPW,), jnp.int32),
          pltpu.VMEM((BPW,), jnp.int32),
          pltpu.VMEM((BPW, F), jnp.float32),
          pltpu.VMEM((BPW, F), jnp.float32),
          pltpu.VMEM((BL,), jnp.float32),
          pltpu.VMEM((BL,), jnp.float32),
          pltpu.VMEM((BPW,), jnp.float32),
          pltpu.SemaphoreType.DMA,
          pltpu.SemaphoreType.DMA,
          pltpu.SemaphoreType.DMA,
          pltpu.SemaphoreType.DMA,
      ],
  )
  return run(users, items, ub, ib, ue, ie)


def kernel(users, items, user_biases, item_biases, user_embeddings,
           item_embeddings):
  users = users.astype(jnp.int32)
  items = items.astype(jnp.int32)
  return _lfm(users, items, user_biases.T, item_biases.T,
              user_embeddings, item_embeddings)


# speedup vs baseline: 1.4659x; 1.4659x over previous
import functools

import jax
import jax.numpy as jnp
from jax import lax
from jax.experimental import pallas as pl
from jax.experimental.pallas import tpu as pltpu
from jax.experimental.pallas import tpu_sc as plsc

NC, NS, L = 2, 16, 16
NW = NC * NS
B = 16384
F = 16
BPW = B // NW
G = BPW // L
BL = BPW * F
CH = 128
NCH = BPW // CH
NR = 1000000 // 8


def _lfm_body(users, items, ub_hbm, ib_hbm, ue_hbm, ie_hbm, out_hbm,
              idx_u, idx_i, ue_s, ie_s, ub_s, ib_s, out_s,
              sem_u, sem_i, sem_ub, sem_ib):
  wid = lax.axis_index("s") * NC + lax.axis_index("c")
  base = wid * BPW

  pltpu.sync_copy(users.at[pl.ds(base, BPW)], idx_u)
  pltpu.sync_copy(items.at[pl.ds(base, BPW)], idx_i)

  lane = lax.iota(jnp.int32, L)

  for c in range(NCH):
    cbase = c * CH

    def fire(j, carry):
      uu = idx_u[pl.ds(cbase + j * L, L)]
      vv = idx_i[pl.ds(cbase + j * L, L)]
      for k in range(L):
        i = j * L + k
        u = uu[k]
        v = vv[k]
        pltpu.async_copy(ue_hbm.at[u], ue_s.at[i], sem_u)
        pltpu.async_copy(ie_hbm.at[v], ie_s.at[i], sem_i)
        pltpu.async_copy(ub_hbm.at[0, pl.ds((u >> 4) * F, F)],
                         ub_s.at[pl.ds((cbase + i) * F, F)], sem_ub)
        pltpu.async_copy(ib_hbm.at[0, pl.ds((v >> 4) * F, F)],
                         ib_s.at[pl.ds((cbase + i) * F, F)], sem_ib)
      return carry

    lax.fori_loop(0, CH // L, fire, 0)

    def drain(i, carry):
      pltpu.make_async_copy(ue_hbm.at[0], ue_s.at[i], sem_u).wait()
      pltpu.make_async_copy(ie_hbm.at[0], ie_s.at[i], sem_i).wait()
      pltpu.make_async_copy(ub_hbm.at[0, pl.ds(0, F)],
                            ub_s.at[pl.ds((cbase + i) * F, F)],
                            sem_ub).wait()
      pltpu.make_async_copy(ib_hbm.at[0, pl.ds(0, F)],
                            ib_s.at[pl.ds((cbase + i) * F, F)],
                            sem_ib).wait()
      return carry

    lax.fori_loop(0, CH, drain, 0)

    for g in range(CH // L):
      off = cbase + g * L
      rows = g * L + lane
      uu = idx_u[pl.ds(off, L)]
      vv = idx_i[pl.ds(off, L)]
      fbase = (off + lane) * F
      acc = (plsc.load_gather(ub_s, [fbase + (uu & 15)]) +
             plsc.load_gather(ib_s, [fbase + (vv & 15)]))
      for f in range(F):
        col = jnp.full((L,), f, jnp.int32)
        acc = acc + (plsc.load_gather(ue_s, [rows, col]) *
                     plsc.load_gather(ie_s, [rows, col]))
      out_s[pl.ds(off, L)] = acc

  pltpu.sync_copy(out_s, out_hbm.at[pl.ds(base, BPW)])


@functools.partial(jax.jit, static_argnames=())
def _lfm(users, items, ub, ib, ue, ie):
  mesh = plsc.VectorSubcoreMesh(
      core_axis_name="c", subcore_axis_name="s",
      num_cores=NC, num_subcores=NS)
  run = pl.kernel(
      _lfm_body,
      out_type=jax.ShapeDtypeStruct((B,), jnp.float32),
      mesh=mesh,
      compiler_params=pltpu.CompilerParams(needs_layout_passes=False),
      scratch_types=[
          pltpu.VMEM((BPW,), jnp.int32),
          pltpu.VMEM((BPW,), jnp.int32),
          pltpu.VMEM((CH, F), jnp.float32),
          pltpu.VMEM((CH, F), jnp.float32),
          pltpu.VMEM((BL,), jnp.float32),
          pltpu.VMEM((BL,), jnp.float32),
          pltpu.VMEM((BPW,), jnp.float32),
          pltpu.SemaphoreType.DMA,
          pltpu.SemaphoreType.DMA,
          pltpu.SemaphoreType.DMA,
          pltpu.SemaphoreType.DMA,
      ],
  )
  return run(users, items, ub, ib, ue, ie)


def kernel(users, items, user_biases, item_biases, user_embeddings,
           item_embeddings):
  users = users.astype(jnp.int32)
  items = items.astype(jnp.int32)
  return _lfm(users, items, user_biases.T, item_biases.T,
              user_embeddings, item_embeddings)
